# baseline (speedup 1.0000x reference)
import jax
import jax.numpy as jnp
from jax import lax
from jax.experimental import pallas as pl
from jax.experimental.pallas import tpu as pltpu
from jax.experimental.pallas import tpu_sc as plsc

N_NODES = 1000000
N_RELS = 1000
EMB = 64
BATCH = 16384
P_EPS = 1e-12

_NC = 2
_NS = 16
_NW = _NC * _NS
_BPW = BATCH // _NW
_CH = 32
_ROUNDS = _BPW // _CH


def _tec_body(head_hbm, rel_hbm, tail_hbm, node_hbm, relemb_hbm, dummy_hbm,
              out_hbm, hidx_v, ridx_v, tidx_v, hp_v, rp_v, tp_v,
              hbuf0, hbuf1, rbuf0, rbuf1, tbuf0, tbuf1, out_v, sem0, sem1):
    wid = lax.axis_index("s") * _NC + lax.axis_index("c")
    base = wid * _BPW

    pltpu.sync_copy(head_hbm.at[pl.ds(base, _BPW)], hidx_v)
    pltpu.sync_copy(rel_hbm.at[pl.ds(base, _BPW)], ridx_v)
    pltpu.sync_copy(tail_hbm.at[pl.ds(base, _BPW)], tidx_v)

    idx_refs = (hidx_v, ridx_v, tidx_v)
    p_refs = (hp_v, rp_v, tp_v)
    bufs = ((hbuf0, rbuf0, tbuf0), (hbuf1, rbuf1, tbuf1))
    sems = (sem0, sem1)
    tables = (node_hbm, relemb_hbm, node_hbm)

    def issue(r, b):
        rb = r * _CH
        for iv, pv in zip(idx_refs, p_refs):
            for k in range(_CH // 16):
                pv[pl.ds(k * 16, 16)] = iv[pl.ds(rb + k * 16, 16)] >> 1
        for tab, pv, dst in zip(tables, p_refs, bufs[b]):
            pltpu.make_async_copy(tab.at[pv], dst, sems[b]).start()

    def drain(b):
        for d in bufs[b]:
            pltpu.make_async_copy(dummy_hbm, d, sems[b]).wait()

    lane = lax.iota(jnp.int32, 16)
    perms = [lane ^ sh for sh in (8, 4, 2, 1)]

    def lanesum(x):
        for p in perms:
            x = x + x.at[p].get(mode="promise_in_bounds")
        return x

    def compute(r, b):
        hb, rb_, tb = bufs[b]
        rbase = r * _CH
        for g in range(_CH // 16):
            par_h = (hidx_v[pl.ds(rbase + g * 16, 16)] & 1) * EMB
            par_r = (ridx_v[pl.ds(rbase + g * 16, 16)] & 1) * EMB
            par_t = (tidx_v[pl.ds(rbase + g * 16, 16)] & 1) * EMB
            s_acc = jnp.zeros((16,), jnp.float32)
            c_acc = jnp.ones((16,), jnp.float32)
            for j in range(16):
                o = g * 16 + j
                jf = jnp.full((16,), j, jnp.int32)
                of = jnp.full((16,), o, jnp.int32)
                ph = par_h.at[jf].get(mode="promise_in_bounds")
                pr = par_r.at[jf].get(mode="promise_in_bounds")
                pt = par_t.at[jf].get(mode="promise_in_bounds")
                h = [plsc.load_gather(hb, [of, ph + (c * 16 + lane)])
                     for c in range(4)]
                rr = [plsc.load_gather(rb_, [of, pr + (c * 16 + lane)])
                      for c in range(4)]
                t = [plsc.load_gather(tb, [of, pt + (c * 16 + lane)])
                     for c in range(4)]
                na = (jnp.abs(h[0]) + jnp.abs(h[1])
                      + jnp.abs(h[2]) + jnp.abs(h[3]))
                nb = (jnp.abs(t[0]) + jnp.abs(t[1])
                      + jnp.abs(t[2]) + jnp.abs(t[3]))
                nh = jnp.maximum(lanesum(na), P_EPS)
                nt = jnp.maximum(lanesum(nb), P_EPS)
                c = nh * nt
                acc = jnp.zeros((16,), jnp.float32)
                for k in range(4):
                    acc = acc + jnp.abs(h[k] * nt + rr[k] * c - t[k] * nh)
                s = lanesum(acc)
                s_acc = jnp.where(lane == j, s, s_acc)
                c_acc = jnp.where(lane == j, c, c_acc)
            out_v[pl.ds(rbase + g * 16, 16)] = s_acc / c_acc

    issue(0, 0)

    def outer(rr, _):
        for b in range(2):
            r = rr * 2 + b

            @pl.when(r + 1 < _ROUNDS)
            def _issue():
                issue(r + 1, 1 - b)

            drain(b)
            compute(r, b)
        return _

    lax.fori_loop(0, _ROUNDS // 2, outer, None)
    pltpu.sync_copy(out_v, out_hbm.at[pl.ds(base, _BPW)])


@jax.jit
def kernel(head_index, rel_type, tail_index, node_emb, rel_emb):
    node_p = node_emb.reshape(N_NODES // 2, 2 * EMB)
    rel_p = rel_emb.reshape(N_RELS // 2, 2 * EMB)
    mesh = plsc.VectorSubcoreMesh(core_axis_name="c", subcore_axis_name="s")
    f = pl.kernel(
        _tec_body,
        out_type=jax.ShapeDtypeStruct((BATCH,), jnp.float32),
        mesh=mesh,
        compiler_params=pltpu.CompilerParams(
            use_tc_tiling_on_sc=True, needs_layout_passes=False),
        scratch_types=[
            pltpu.VMEM((_BPW,), jnp.int32),
            pltpu.VMEM((_BPW,), jnp.int32),
            pltpu.VMEM((_BPW,), jnp.int32),
            pltpu.VMEM((_CH,), jnp.int32),
            pltpu.VMEM((_CH,), jnp.int32),
            pltpu.VMEM((_CH,), jnp.int32),
            pltpu.VMEM((_CH, 2 * EMB), jnp.float32),
            pltpu.VMEM((_CH, 2 * EMB), jnp.float32),
            pltpu.VMEM((_CH, 2 * EMB), jnp.float32),
            pltpu.VMEM((_CH, 2 * EMB), jnp.float32),
            pltpu.VMEM((_CH, 2 * EMB), jnp.float32),
            pltpu.VMEM((_CH, 2 * EMB), jnp.float32),
            pltpu.VMEM((_BPW,), jnp.float32),
            pltpu.SemaphoreType.DMA,
            pltpu.SemaphoreType.DMA,
        ],
    )
    dummy = jnp.zeros((_CH, 2 * EMB), jnp.float32)
    return f(head_index, rel_type, tail_index, node_p, rel_p, dummy)


# speedup vs baseline: 1.1394x; 1.1394x over previous
import jax
import jax.numpy as jnp
from jax import lax
from jax.experimental import pallas as pl
from jax.experimental.pallas import tpu as pltpu
from jax.experimental.pallas import tpu_sc as plsc

N_NODES = 1000000
N_RELS = 1000
EMB = 64
BATCH = 16384
P_EPS = 1e-12

_NC = 2
_NS = 16
_NW = _NC * _NS
_BPW = BATCH // _NW
_CH = 32
_ROUNDS = _BPW // _CH


def _tec_body(head_hbm, rel_hbm, tail_hbm, node_hbm, relemb_hbm, dummy_hbm,
              out_hbm, hidx_v, ridx_v, tidx_v, hp0, rp0, tp0, hp1, rp1, tp1,
              hbuf0, hbuf1, rbuf0, rbuf1, tbuf0, tbuf1, out_v, sem0, sem1):
    wid = lax.axis_index("s") * _NC + lax.axis_index("c")
    base = wid * _BPW

    pltpu.sync_copy(head_hbm.at[pl.ds(base, _BPW)], hidx_v)
    pltpu.sync_copy(rel_hbm.at[pl.ds(base, _BPW)], ridx_v)
    pltpu.sync_copy(tail_hbm.at[pl.ds(base, _BPW)], tidx_v)

    idx_refs = (hidx_v, ridx_v, tidx_v)
    p_refs = ((hp0, rp0, tp0), (hp1, rp1, tp1))
    bufs = ((hbuf0, rbuf0, tbuf0), (hbuf1, rbuf1, tbuf1))
    sems = (sem0, sem1)
    tables = (node_hbm, relemb_hbm, node_hbm)

    def issue(r, b):
        rb = r * _CH
        for iv, pv in zip(idx_refs, p_refs[b]):
            for k in range(_CH // 16):
                pv[pl.ds(k * 16, 16)] = iv[pl.ds(rb + k * 16, 16)]
        for tab, pv, dst in zip(tables, p_refs[b], bufs[b]):
            pltpu.make_async_copy(tab.at[pv], dst, sems[b]).start()

    def drain(b):
        for d in bufs[b]:
            pltpu.make_async_copy(dummy_hbm, d, sems[b]).wait()

    lane = lax.iota(jnp.int32, 16)
    perms = [lane ^ sh for sh in (8, 4, 2, 1)]

    def lanesum(x):
        for p in perms:
            x = x + x.at[p].get(mode="promise_in_bounds")
        return x

    def compute(r, b):
        hb, rb_, tb = bufs[b]
        rbase = r * _CH
        for g in range(_CH // 16):
            s_acc = jnp.zeros((16,), jnp.float32)
            c_acc = jnp.ones((16,), jnp.float32)
            for j in range(16):
                o = g * 16 + j
                h = [hb[o, pl.ds(c * 16, 16)] for c in range(4)]
                rr = [rb_[o, pl.ds(c * 16, 16)] for c in range(4)]
                t = [tb[o, pl.ds(c * 16, 16)] for c in range(4)]
                na = (jnp.abs(h[0]) + jnp.abs(h[1])
                      + jnp.abs(h[2]) + jnp.abs(h[3]))
                nb = (jnp.abs(t[0]) + jnp.abs(t[1])
                      + jnp.abs(t[2]) + jnp.abs(t[3]))
                nh = jnp.maximum(lanesum(na), P_EPS)
                nt = jnp.maximum(lanesum(nb), P_EPS)
                c = nh * nt
                acc = jnp.zeros((16,), jnp.float32)
                for k in range(4):
                    acc = acc + jnp.abs(h[k] * nt + rr[k] * c - t[k] * nh)
                s = lanesum(acc)
                s_acc = jnp.where(lane == j, s, s_acc)
                c_acc = jnp.where(lane == j, c, c_acc)
            out_v[pl.ds(rbase + g * 16, 16)] = s_acc / c_acc

    issue(0, 0)

    def outer(rr, _):
        for b in range(2):
            r = rr * 2 + b

            @pl.when(r + 1 < _ROUNDS)
            def _issue():
                issue(r + 1, 1 - b)

            drain(b)
            compute(r, b)
        return _

    lax.fori_loop(0, _ROUNDS // 2, outer, None)
    pltpu.sync_copy(out_v, out_hbm.at[pl.ds(base, _BPW)])


@jax.jit
def kernel(head_index, rel_type, tail_index, node_emb, rel_emb):
    node_p = jnp.pad(node_emb, ((0, 0), (0, EMB)))
    rel_p = jnp.pad(rel_emb, ((0, 0), (0, EMB)))
    mesh = plsc.VectorSubcoreMesh(core_axis_name="c", subcore_axis_name="s")
    f = pl.kernel(
        _tec_body,
        out_type=jax.ShapeDtypeStruct((BATCH,), jnp.float32),
        mesh=mesh,
        compiler_params=pltpu.CompilerParams(
            use_tc_tiling_on_sc=True, needs_layout_passes=False),
        scratch_types=[
            pltpu.VMEM((_BPW,), jnp.int32),
            pltpu.VMEM((_BPW,), jnp.int32),
            pltpu.VMEM((_BPW,), jnp.int32),
            pltpu.VMEM((_CH,), jnp.int32),
            pltpu.VMEM((_CH,), jnp.int32),
            pltpu.VMEM((_CH,), jnp.int32),
            pltpu.VMEM((_CH,), jnp.int32),
            pltpu.VMEM((_CH,), jnp.int32),
            pltpu.VMEM((_CH,), jnp.int32),
            pltpu.VMEM((_CH, 2 * EMB), jnp.float32),
            pltpu.VMEM((_CH, 2 * EMB), jnp.float32),
            pltpu.VMEM((_CH, 2 * EMB), jnp.float32),
            pltpu.VMEM((_CH, 2 * EMB), jnp.float32),
            pltpu.VMEM((_CH, 2 * EMB), jnp.float32),
            pltpu.VMEM((_CH, 2 * EMB), jnp.float32),
            pltpu.VMEM((_BPW,), jnp.float32),
            pltpu.SemaphoreType.DMA,
            pltpu.SemaphoreType.DMA,
        ],
    )
    dummy = jnp.zeros((_CH, 2 * EMB), jnp.float32)
    return f(head_index, rel_type, tail_index, node_p, rel_p, dummy)
